# Initial kernel scaffold; baseline (speedup 1.0000x reference)
import functools

import jax
import jax.numpy as jnp
from jax.experimental import pallas as pl
from jax.experimental.pallas import tpu as pltpu

N_NODES = 50000
ROW_BLK = 2000


def _row_grid(n):
    return n // ROW_BLK


def _leaky(t):
    return jnp.where(t > 0, t, 0.2 * t)


def _gin_mlp_body(x_ref, agg_ref, w1_ref, b1_ref, w2_ref, b2_ref, o_ref,
                  *, relu_out):
    h = x_ref[...] + agg_ref[...]
    h = jnp.maximum(jnp.dot(h, w1_ref[...],
                            preferred_element_type=jnp.float32) + b1_ref[...],
                    0.0)
    o = jnp.dot(h, w2_ref[...], preferred_element_type=jnp.float32) + b2_ref[...]
    if relu_out:
        o = jnp.maximum(o, 0.0)
    o_ref[...] = o


def _gin_mlp(x, agg, w1, b1, w2, b2, relu_out):
    n, din = x.shape
    dout = w2.shape[1]
    blk = lambda d: pl.BlockSpec((ROW_BLK, d), lambda i: (i, 0))
    full = lambda a: pl.BlockSpec(a.shape, lambda i: tuple(0 for _ in a.shape))
    return pl.pallas_call(
        functools.partial(_gin_mlp_body, relu_out=relu_out),
        grid=(_row_grid(n),),
        in_specs=[blk(din), blk(din), full(w1), full(b1), full(w2), full(b2)],
        out_specs=blk(dout),
        out_shape=jax.ShapeDtypeStruct((n, dout), jnp.float32),
    )(x, agg, w1, b1, w2, b2)


def _gat_pre_body(x_ref, avs_ref, avd_ref, asrc_ref, adst_ref, eself_ref):
    x = x_ref[...]
    a_s = jnp.dot(x, avs_ref[...], preferred_element_type=jnp.float32)
    a_d = jnp.dot(x, avd_ref[...], preferred_element_type=jnp.float32)
    asrc_ref[...] = a_s
    adst_ref[...] = a_d
    eself_ref[...] = _leaky(a_s + a_d)


def _gat_pre(x, av_s, av_d):
    n, din = x.shape
    blk2 = pl.BlockSpec((ROW_BLK, din), lambda i: (i, 0))
    blk1 = pl.BlockSpec((ROW_BLK,), lambda i: (i,))
    full = pl.BlockSpec((din,), lambda i: (0,))
    return pl.pallas_call(
        _gat_pre_body,
        grid=(_row_grid(n),),
        in_specs=[blk2, full, full],
        out_specs=[blk1, blk1, blk1],
        out_shape=[jax.ShapeDtypeStruct((n,), jnp.float32)] * 3,
    )(x, av_s, av_d)


def _gat1_post_body(wsum_ref, denom_ref, eself_ref, m_ref, x_ref, w_ref,
                    b_ref, o_ref):
    ee_self = jnp.exp(eself_ref[...] - m_ref[...])
    num = wsum_ref[...] + ee_self[:, None] * x_ref[...]
    den = denom_ref[...] + ee_self + 1e-16
    agg = num / den[:, None]
    o = jnp.dot(agg, w_ref[...], preferred_element_type=jnp.float32) + b_ref[...]
    o_ref[...] = jnp.maximum(o, 0.0)


def _gat1_post(wsum, denom, e_self, m, x, w, b):
    n, din = x.shape
    dout = w.shape[1]
    blk2 = lambda d: pl.BlockSpec((ROW_BLK, d), lambda i: (i, 0))
    blk1 = pl.BlockSpec((ROW_BLK,), lambda i: (i,))
    full = lambda a: pl.BlockSpec(a.shape, lambda i: tuple(0 for _ in a.shape))
    return pl.pallas_call(
        _gat1_post_body,
        grid=(_row_grid(n),),
        in_specs=[blk2(din), blk1, blk1, blk1, blk2(din), full(w), full(b)],
        out_specs=blk2(dout),
        out_shape=jax.ShapeDtypeStruct((n, dout), jnp.float32),
    )(wsum, denom, e_self, m, x, w, b)


def _gat2_post_body(wsum_ref, denom_ref, eself_ref, m_ref, g_ref, b_ref, o_ref):
    ee_self = jnp.exp(eself_ref[...] - m_ref[...])
    num = wsum_ref[...] + ee_self[:, None] * g_ref[...]
    den = denom_ref[...] + ee_self + 1e-16
    o_ref[...] = num / den[:, None] + b_ref[...]


def _gat2_post(wsum, denom, e_self, m, g, b):
    n, d = g.shape
    blk2 = pl.BlockSpec((ROW_BLK, d), lambda i: (i, 0))
    blk1 = pl.BlockSpec((ROW_BLK,), lambda i: (i,))
    full = pl.BlockSpec(b.shape, lambda i: (0,))
    return pl.pallas_call(
        _gat2_post_body,
        grid=(_row_grid(n),),
        in_specs=[blk2, blk1, blk1, blk1, blk2, full],
        out_specs=blk2,
        out_shape=jax.ShapeDtypeStruct((n, d), jnp.float32),
    )(wsum, denom, e_self, m, g, b)


def _matmul_body(x_ref, w_ref, o_ref):
    o_ref[...] = jnp.dot(x_ref[...], w_ref[...],
                         preferred_element_type=jnp.float32)


def _matmul(x, w):
    n, din = x.shape
    dout = w.shape[1]
    return pl.pallas_call(
        _matmul_body,
        grid=(_row_grid(n),),
        in_specs=[pl.BlockSpec((ROW_BLK, din), lambda i: (i, 0)),
                  pl.BlockSpec((din, dout), lambda i: (0, 0))],
        out_specs=pl.BlockSpec((ROW_BLK, dout), lambda i: (i, 0)),
        out_shape=jax.ShapeDtypeStruct((n, dout), jnp.float32),
    )(x, w)


def _seg_sum_rows(x, src, dst, n):
    return jax.ops.segment_sum(x[src], dst, num_segments=n)


def _edge_logits(asrc, adst, src, dst):
    return _leaky(asrc[src] + adst[dst])


def _seg_max(e, dst, n):
    m = jax.ops.segment_max(e, dst, num_segments=n)
    return jnp.where(jnp.isfinite(m), m, -jnp.inf)


def _edge_exp_segsum(e, m, dst, n):
    ee = jnp.exp(e - m[dst])
    denom = jax.ops.segment_sum(ee, dst, num_segments=n)
    return ee, denom


def _seg_weighted_sum(x, ee, src, dst, n):
    return jax.ops.segment_sum(x[src] * ee[:, None], dst, num_segments=n)


def kernel(x, edge_index, edge_attr, gin1_w1, gin1_b1, gin1_w2, gin1_b2,
           gin2_w1, gin2_b1, gin2_w2, gin2_b2, gat1_w, gat1_asrc, gat1_adst,
           gat1_b, gat2_w, gat2_asrc, gat2_adst, gat2_b):
    del edge_attr
    n = x.shape[0]
    src = edge_index[0]
    dst = edge_index[1]

    agg = _seg_sum_rows(x, src, dst, n)
    h = _gin_mlp(x, agg, gin1_w1, gin1_b1, gin1_w2, gin1_b2, relu_out=True)

    agg = _seg_sum_rows(h, src, dst, n)
    h = _gin_mlp(h, agg, gin2_w1, gin2_b1, gin2_w2, gin2_b2, relu_out=False)

    av_s = gat1_w @ gat1_asrc
    av_d = gat1_w @ gat1_adst
    asrc, adst, e_self = _gat_pre(h, av_s, av_d)
    e = _edge_logits(asrc, adst, src, dst)
    m = jnp.maximum(_seg_max(e, dst, n), e_self)
    ee, denom = _edge_exp_segsum(e, m, dst, n)
    wsum = _seg_weighted_sum(h, ee, src, dst, n)
    h = _gat1_post(wsum, denom, e_self, m, h, gat1_w, gat1_b)

    g = _matmul(h, gat2_w)
    asrc, adst, e_self = _gat_pre(g, gat2_asrc, gat2_adst)
    e = _edge_logits(asrc, adst, src, dst)
    m = jnp.maximum(_seg_max(e, dst, n), e_self)
    ee, denom = _edge_exp_segsum(e, m, dst, n)
    wsum = _seg_weighted_sum(g, ee, src, dst, n)
    return _gat2_post(wsum, denom, e_self, m, g, gat2_b)


# speedup vs baseline: 1.4728x; 1.4728x over previous
import functools

import jax
import jax.numpy as jnp
from jax.experimental import pallas as pl
from jax.experimental.pallas import tpu as pltpu

N_NODES = 50000
ROW_BLK = 2048


def _row_grid(n):
    return pl.cdiv(n, ROW_BLK)


def _leaky(t):
    return jnp.where(t > 0, t, 0.2 * t)


def _gin_mlp_body(x_ref, agg_ref, w1_ref, b1_ref, w2_ref, b2_ref, o_ref,
                  *, relu_out):
    h = x_ref[...] + agg_ref[...]
    h = jnp.maximum(jnp.dot(h, w1_ref[...],
                            preferred_element_type=jnp.float32) + b1_ref[...],
                    0.0)
    o = jnp.dot(h, w2_ref[...], preferred_element_type=jnp.float32) + b2_ref[...]
    if relu_out:
        o = jnp.maximum(o, 0.0)
    o_ref[...] = o


def _gin_mlp(x, agg, w1, b1, w2, b2, relu_out):
    n, din = x.shape
    dout = w2.shape[1]
    blk = lambda d: pl.BlockSpec((ROW_BLK, d), lambda i: (i, 0))
    full = lambda a: pl.BlockSpec(a.shape, lambda i: tuple(0 for _ in a.shape))
    return pl.pallas_call(
        functools.partial(_gin_mlp_body, relu_out=relu_out),
        grid=(_row_grid(n),),
        in_specs=[blk(din), blk(din), full(w1), full(b1), full(w2), full(b2)],
        out_specs=blk(dout),
        out_shape=jax.ShapeDtypeStruct((n, dout), jnp.float32),
    )(x, agg, w1, b1, w2, b2)


def _gat_pre_body(x_ref, avs_ref, avd_ref, asrc_ref, adst_ref, eself_ref):
    x = x_ref[...]
    a_s = jnp.dot(x, avs_ref[...], preferred_element_type=jnp.float32)
    a_d = jnp.dot(x, avd_ref[...], preferred_element_type=jnp.float32)
    asrc_ref[...] = a_s
    adst_ref[...] = a_d
    eself_ref[...] = _leaky(a_s + a_d)


def _gat_pre(x, av_s, av_d):
    n, din = x.shape
    blk2 = pl.BlockSpec((ROW_BLK, din), lambda i: (i, 0))
    blk1 = pl.BlockSpec((ROW_BLK,), lambda i: (i,))
    full = pl.BlockSpec((din,), lambda i: (0,))
    return pl.pallas_call(
        _gat_pre_body,
        grid=(_row_grid(n),),
        in_specs=[blk2, full, full],
        out_specs=[blk1, blk1, blk1],
        out_shape=[jax.ShapeDtypeStruct((n,), jnp.float32)] * 3,
    )(x, av_s, av_d)


def _gat1_post_body(wsum_ref, denom_ref, eself_ref, m_ref, x_ref, w_ref,
                    b_ref, o_ref):
    ee_self = jnp.exp(eself_ref[...] - m_ref[...])
    num = wsum_ref[...] + ee_self[:, None] * x_ref[...]
    den = denom_ref[...] + ee_self + 1e-16
    agg = num / den[:, None]
    o = jnp.dot(agg, w_ref[...], preferred_element_type=jnp.float32) + b_ref[...]
    o_ref[...] = jnp.maximum(o, 0.0)


def _gat1_post(wsum, denom, e_self, m, x, w, b):
    n, din = x.shape
    dout = w.shape[1]
    blk2 = lambda d: pl.BlockSpec((ROW_BLK, d), lambda i: (i, 0))
    blk1 = pl.BlockSpec((ROW_BLK,), lambda i: (i,))
    full = lambda a: pl.BlockSpec(a.shape, lambda i: tuple(0 for _ in a.shape))
    return pl.pallas_call(
        _gat1_post_body,
        grid=(_row_grid(n),),
        in_specs=[blk2(din), blk1, blk1, blk1, blk2(din), full(w), full(b)],
        out_specs=blk2(dout),
        out_shape=jax.ShapeDtypeStruct((n, dout), jnp.float32),
    )(wsum, denom, e_self, m, x, w, b)


def _gat2_post_body(wsum_ref, denom_ref, eself_ref, m_ref, g_ref, b_ref, o_ref):
    ee_self = jnp.exp(eself_ref[...] - m_ref[...])
    num = wsum_ref[...] + ee_self[:, None] * g_ref[...]
    den = denom_ref[...] + ee_self + 1e-16
    o_ref[...] = num / den[:, None] + b_ref[...]


def _gat2_post(wsum, denom, e_self, m, g, b):
    n, d = g.shape
    blk2 = pl.BlockSpec((ROW_BLK, d), lambda i: (i, 0))
    blk1 = pl.BlockSpec((ROW_BLK,), lambda i: (i,))
    full = pl.BlockSpec(b.shape, lambda i: (0,))
    return pl.pallas_call(
        _gat2_post_body,
        grid=(_row_grid(n),),
        in_specs=[blk2, blk1, blk1, blk1, blk2, full],
        out_specs=blk2,
        out_shape=jax.ShapeDtypeStruct((n, d), jnp.float32),
    )(wsum, denom, e_self, m, g, b)


def _matmul_body(x_ref, w_ref, o_ref):
    o_ref[...] = jnp.dot(x_ref[...], w_ref[...],
                         preferred_element_type=jnp.float32)


def _matmul(x, w):
    n, din = x.shape
    dout = w.shape[1]
    return pl.pallas_call(
        _matmul_body,
        grid=(_row_grid(n),),
        in_specs=[pl.BlockSpec((ROW_BLK, din), lambda i: (i, 0)),
                  pl.BlockSpec((din, dout), lambda i: (0, 0))],
        out_specs=pl.BlockSpec((ROW_BLK, dout), lambda i: (i, 0)),
        out_shape=jax.ShapeDtypeStruct((n, dout), jnp.float32),
    )(x, w)


def _seg_sum_rows(x, src, dst, n):
    return jax.ops.segment_sum(x[src], dst, num_segments=n)


def _edge_logits(asrc, adst, src, dst):
    return _leaky(asrc[src] + adst[dst])


def _seg_max(e, dst, n):
    m = jax.ops.segment_max(e, dst, num_segments=n)
    return jnp.where(jnp.isfinite(m), m, -jnp.inf)


def _edge_exp_segsum(e, m, dst, n):
    ee = jnp.exp(e - m[dst])
    denom = jax.ops.segment_sum(ee, dst, num_segments=n)
    return ee, denom


def _seg_weighted_sum(x, ee, src, dst, n):
    return jax.ops.segment_sum(x[src] * ee[:, None], dst, num_segments=n)


def kernel(x, edge_index, edge_attr, gin1_w1, gin1_b1, gin1_w2, gin1_b2,
           gin2_w1, gin2_b1, gin2_w2, gin2_b2, gat1_w, gat1_asrc, gat1_adst,
           gat1_b, gat2_w, gat2_asrc, gat2_adst, gat2_b):
    del edge_attr
    n = x.shape[0]
    src = edge_index[0]
    dst = edge_index[1]

    agg = _seg_sum_rows(x, src, dst, n)
    h = _gin_mlp(x, agg, gin1_w1, gin1_b1, gin1_w2, gin1_b2, relu_out=True)

    agg = _seg_sum_rows(h, src, dst, n)
    h = _gin_mlp(h, agg, gin2_w1, gin2_b1, gin2_w2, gin2_b2, relu_out=False)

    av_s = gat1_w @ gat1_asrc
    av_d = gat1_w @ gat1_adst
    asrc, adst, e_self = _gat_pre(h, av_s, av_d)
    e = _edge_logits(asrc, adst, src, dst)
    m = jnp.maximum(_seg_max(e, dst, n), e_self)
    ee, denom = _edge_exp_segsum(e, m, dst, n)
    wsum = _seg_weighted_sum(h, ee, src, dst, n)
    h = _gat1_post(wsum, denom, e_self, m, h, gat1_w, gat1_b)

    g = _matmul(h, gat2_w)
    asrc, adst, e_self = _gat_pre(g, gat2_asrc, gat2_adst)
    e = _edge_logits(asrc, adst, src, dst)
    m = jnp.maximum(_seg_max(e, dst, n), e_self)
    ee, denom = _edge_exp_segsum(e, m, dst, n)
    wsum = _seg_weighted_sum(g, ee, src, dst, n)
    return _gat2_post(wsum, denom, e_self, m, g, gat2_b)


# speedup vs baseline: 1.5994x; 1.0860x over previous
import functools

import jax
import jax.numpy as jnp
from jax import lax
from jax.experimental import pallas as pl
from jax.experimental.pallas import tpu as pltpu
from jax.experimental.pallas import tpu_sc as plsc

N_NODES = 50000
ROW_BLK = 2048

SC_CORES = 2
SC_TILES = 16
SC_WORKERS = SC_CORES * SC_TILES
EDGE_W = 128
E_EDGES = 800000
ROWS_PER_TILE = 200
EPAD = ROWS_PER_TILE * EDGE_W * SC_WORKERS
NACC = 50048
STRIPE = NACC // SC_TILES

@functools.cache
def _sc_mesh():
    return plsc.VectorSubcoreMesh(core_axis_name="c", subcore_axis_name="s")


def _row_grid(n):
    return pl.cdiv(n, ROW_BLK)


def _leaky(t):
    return jnp.where(t > 0, t, 0.2 * t)


def _gin_mlp_body(x_ref, parts_ref, w1_ref, b1_ref, w2_ref, b2_ref, o_ref,
                  *, relu_out, din):
    p = parts_ref[...]
    nchunk = p.shape[1]
    agg = jnp.concatenate([p[0, k] + p[1, k] for k in range(nchunk)],
                          axis=-1)[:, :din]
    h = x_ref[...] + agg
    h = jnp.maximum(jnp.dot(h, w1_ref[...],
                            preferred_element_type=jnp.float32) + b1_ref[...],
                    0.0)
    o = jnp.dot(h, w2_ref[...], preferred_element_type=jnp.float32) + b2_ref[...]
    if relu_out:
        o = jnp.maximum(o, 0.0)
    o_ref[...] = o


def _gin_mlp(x, parts, w1, b1, w2, b2, relu_out):
    n, din = x.shape
    dout = w2.shape[1]
    _, nchunk, _, c_w = parts.shape
    blk = lambda d: pl.BlockSpec((ROW_BLK, d), lambda i: (i, 0))
    pblk = pl.BlockSpec((SC_CORES, nchunk, ROW_BLK, c_w),
                        lambda i: (0, 0, i, 0))
    full = lambda a: pl.BlockSpec(a.shape, lambda i: tuple(0 for _ in a.shape))
    return pl.pallas_call(
        functools.partial(_gin_mlp_body, relu_out=relu_out, din=din),
        grid=(_row_grid(n),),
        in_specs=[blk(din), pblk, full(w1), full(b1), full(w2), full(b2)],
        out_specs=blk(dout),
        out_shape=jax.ShapeDtypeStruct((n, dout), jnp.float32),
    )(x, parts, w1, b1, w2, b2)


def _gat_pre_body(x_ref, avs_ref, avd_ref, asrc_ref, adst_ref, eself_ref):
    x = x_ref[...]
    a_s = jnp.dot(x, avs_ref[...], preferred_element_type=jnp.float32)
    a_d = jnp.dot(x, avd_ref[...], preferred_element_type=jnp.float32)
    asrc_ref[...] = a_s
    adst_ref[...] = a_d
    eself_ref[...] = _leaky(a_s + a_d)


def _gat_pre(x, av_s, av_d):
    n, din = x.shape
    blk2 = pl.BlockSpec((ROW_BLK, din), lambda i: (i, 0))
    blk1 = pl.BlockSpec((ROW_BLK,), lambda i: (i,))
    full = pl.BlockSpec((din,), lambda i: (0,))
    return pl.pallas_call(
        _gat_pre_body,
        grid=(_row_grid(n),),
        in_specs=[blk2, full, full],
        out_specs=[blk1, blk1, blk1],
        out_shape=[jax.ShapeDtypeStruct((n,), jnp.float32)] * 3,
    )(x, av_s, av_d)


def _gat1_post_body(wsum_ref, denom_ref, eself_ref, m_ref, x_ref, w_ref,
                    b_ref, o_ref):
    ee_self = jnp.exp(eself_ref[...] - m_ref[...])
    num = wsum_ref[...] + ee_self[:, None] * x_ref[...]
    den = denom_ref[...] + ee_self + 1e-16
    agg = num / den[:, None]
    o = jnp.dot(agg, w_ref[...], preferred_element_type=jnp.float32) + b_ref[...]
    o_ref[...] = jnp.maximum(o, 0.0)


def _gat1_post(wsum, denom, e_self, m, x, w, b):
    n, din = x.shape
    dout = w.shape[1]
    blk2 = lambda d: pl.BlockSpec((ROW_BLK, d), lambda i: (i, 0))
    blk1 = pl.BlockSpec((ROW_BLK,), lambda i: (i,))
    full = lambda a: pl.BlockSpec(a.shape, lambda i: tuple(0 for _ in a.shape))
    return pl.pallas_call(
        _gat1_post_body,
        grid=(_row_grid(n),),
        in_specs=[blk2(din), blk1, blk1, blk1, blk2(din), full(w), full(b)],
        out_specs=blk2(dout),
        out_shape=jax.ShapeDtypeStruct((n, dout), jnp.float32),
    )(wsum, denom, e_self, m, x, w, b)


def _gat2_post_body(wsum_ref, denom_ref, eself_ref, m_ref, g_ref, b_ref, o_ref):
    ee_self = jnp.exp(eself_ref[...] - m_ref[...])
    num = wsum_ref[...] + ee_self[:, None] * g_ref[...]
    den = denom_ref[...] + ee_self + 1e-16
    o_ref[...] = num / den[:, None] + b_ref[...]


def _gat2_post(wsum, denom, e_self, m, g, b):
    n, d = g.shape
    blk2 = pl.BlockSpec((ROW_BLK, d), lambda i: (i, 0))
    blk1 = pl.BlockSpec((ROW_BLK,), lambda i: (i,))
    full = pl.BlockSpec(b.shape, lambda i: (0,))
    return pl.pallas_call(
        _gat2_post_body,
        grid=(_row_grid(n),),
        in_specs=[blk2, blk1, blk1, blk1, blk2, full],
        out_specs=blk2,
        out_shape=jax.ShapeDtypeStruct((n, d), jnp.float32),
    )(wsum, denom, e_self, m, g, b)


def _matmul_body(x_ref, w_ref, o_ref):
    o_ref[...] = jnp.dot(x_ref[...], w_ref[...],
                         preferred_element_type=jnp.float32)


def _matmul(x, w):
    n, din = x.shape
    dout = w.shape[1]
    return pl.pallas_call(
        _matmul_body,
        grid=(_row_grid(n),),
        in_specs=[pl.BlockSpec((ROW_BLK, din), lambda i: (i, 0)),
                  pl.BlockSpec((din, dout), lambda i: (0, 0))],
        out_specs=pl.BlockSpec((ROW_BLK, dout), lambda i: (i, 0)),
        out_shape=jax.ShapeDtypeStruct((n, dout), jnp.float32),
    )(x, w)


def _prep_edges(edge_index):
    src = edge_index[0]
    dst = edge_index[1]
    pad = EPAD - E_EDGES
    src_p = jnp.concatenate([src, jnp.zeros((pad,), jnp.int32)])
    dst_p = jnp.concatenate(
        [dst, N_NODES + (jnp.arange(pad, dtype=jnp.int32) % SC_TILES)])
    return src_p.reshape(-1, EDGE_W), dst_p.reshape(-1, EDGE_W)


def _sc_segsum(tables, src2d, dst2d, zeros_nc):
    nchunk = len(tables)
    c_w = tables[0].shape[1]
    out_type = jax.ShapeDtypeStruct((SC_CORES, nchunk, NACC, c_w),
                                    jnp.float32)

    @functools.partial(
        pl.kernel, out_type=out_type, mesh=_sc_mesh(),
        compiler_params=pltpu.CompilerParams(use_tc_tiling_on_sc=False),
        scratch_types=[
            pltpu.VMEM((EDGE_W,), jnp.int32),
            pltpu.VMEM((EDGE_W,), jnp.int32),
            pltpu.VMEM((EDGE_W, c_w), jnp.float32),
            pltpu.VMEM_SHARED((NACC, c_w), jnp.float32),
        ])
    def k(*refs):
        tbls = refs[:nchunk]
        (src_hbm, dst_hbm, zeros_hbm, out_hbm,
         src_blk, dst_blk, rows, acc) = refs[nchunk:]
        cid = lax.axis_index("c")
        sid = lax.axis_index("s")
        wid = cid * SC_TILES + sid
        row0 = wid * ROWS_PER_TILE
        pltpu.sync_copy(zeros_hbm.at[pl.ds(sid * STRIPE, STRIPE)],
                        acc.at[pl.ds(sid * STRIPE, STRIPE)])
        plsc.subcore_barrier()
        for c in range(nchunk):
            @pl.loop(0, ROWS_PER_TILE)
            def _(w):
                pltpu.sync_copy(src_hbm.at[row0 + w], src_blk)
                pltpu.sync_copy(dst_hbm.at[row0 + w], dst_blk)
                pltpu.sync_copy(tbls[c].at[src_blk], rows)
                pltpu.sync_copy(rows, acc.at[dst_blk], add=True)
            plsc.subcore_barrier()
            pltpu.sync_copy(
                acc.at[pl.ds(sid * STRIPE, STRIPE)],
                out_hbm.at[cid, c, pl.ds(sid * STRIPE, STRIPE)])
            if c + 1 < nchunk:
                pltpu.sync_copy(zeros_hbm.at[pl.ds(sid * STRIPE, STRIPE)],
                                acc.at[pl.ds(sid * STRIPE, STRIPE)])
                plsc.subcore_barrier()

    return k(*tables, src2d, dst2d, zeros_nc)


def _seg_sum_rows(x, src, dst, n):
    return jax.ops.segment_sum(x[src], dst, num_segments=n)


def _edge_logits(asrc, adst, src, dst):
    return _leaky(asrc[src] + adst[dst])


def _seg_max(e, dst, n):
    m = jax.ops.segment_max(e, dst, num_segments=n)
    return jnp.where(jnp.isfinite(m), m, -jnp.inf)


def _edge_exp_segsum(e, m, dst, n):
    ee = jnp.exp(e - m[dst])
    denom = jax.ops.segment_sum(ee, dst, num_segments=n)
    return ee, denom


def _seg_weighted_sum(x, ee, src, dst, n):
    return jax.ops.segment_sum(x[src] * ee[:, None], dst, num_segments=n)


def kernel(x, edge_index, edge_attr, gin1_w1, gin1_b1, gin1_w2, gin1_b2,
           gin2_w1, gin2_b1, gin2_w2, gin2_b2, gat1_w, gat1_asrc, gat1_adst,
           gat1_b, gat2_w, gat2_asrc, gat2_adst, gat2_b):
    del edge_attr
    n = x.shape[0]
    src = edge_index[0]
    dst = edge_index[1]
    src2d, dst2d = _prep_edges(edge_index)
    zeros16 = jnp.zeros((NACC, 16), jnp.float32)
    zeros32 = jnp.zeros((NACC, 32), jnp.float32)

    x16 = jnp.pad(x, ((0, 0), (0, 6)))
    parts = _sc_segsum([x16], src2d, dst2d, zeros16)
    h = _gin_mlp(x, parts, gin1_w1, gin1_b1, gin1_w2, gin1_b2, relu_out=True)

    parts = _sc_segsum([h[:, :32], h[:, 32:]], src2d, dst2d, zeros32)
    h = _gin_mlp(h, parts, gin2_w1, gin2_b1, gin2_w2, gin2_b2, relu_out=False)

    av_s = gat1_w @ gat1_asrc
    av_d = gat1_w @ gat1_adst
    asrc, adst, e_self = _gat_pre(h, av_s, av_d)
    e = _edge_logits(asrc, adst, src, dst)
    m = jnp.maximum(_seg_max(e, dst, n), e_self)
    ee, denom = _edge_exp_segsum(e, m, dst, n)
    wsum = _seg_weighted_sum(h, ee, src, dst, n)
    h = _gat1_post(wsum, denom, e_self, m, h, gat1_w, gat1_b)

    g = _matmul(h, gat2_w)
    asrc, adst, e_self = _gat_pre(g, gat2_asrc, gat2_adst)
    e = _edge_logits(asrc, adst, src, dst)
    m = jnp.maximum(_seg_max(e, dst, n), e_self)
    ee, denom = _edge_exp_segsum(e, m, dst, n)
    wsum = _seg_weighted_sum(g, ee, src, dst, n)
    return _gat2_post(wsum, denom, e_self, m, g, gat2_b)


# speedup vs baseline: 7.5673x; 4.7312x over previous
import functools

import jax
import jax.numpy as jnp
from jax import lax
from jax.experimental import pallas as pl
from jax.experimental.pallas import tpu as pltpu
from jax.experimental.pallas import tpu_sc as plsc

N_NODES = 50000
ROW_BLK = 2048

SC_CORES = 2
SC_TILES = 16
SC_WORKERS = SC_CORES * SC_TILES
EDGE_W = 128
E_EDGES = 800000
ROWS_PER_TILE = 200
EPAD = ROWS_PER_TILE * EDGE_W * SC_WORKERS
NACC = 50048
STRIPE = NACC // SC_TILES

@functools.cache
def _sc_mesh():
    return plsc.VectorSubcoreMesh(core_axis_name="c", subcore_axis_name="s")


def _row_grid(n):
    return pl.cdiv(n, ROW_BLK)


def _lane_bcast(v, l):
    idx = jnp.full((16, 1), l, jnp.int32)
    dnums = lax.GatherDimensionNumbers(
        offset_dims=(), collapsed_slice_dims=(0,), start_index_map=(0,))
    return lax.gather(v, idx, dnums, (1,),
                      mode=lax.GatherScatterMode.PROMISE_IN_BOUNDS)


def _leaky(t):
    return jnp.where(t > 0, t, 0.2 * t)


def _gin_mlp_body(x_ref, parts_ref, w1_ref, b1_ref, w2_ref, b2_ref, o_ref,
                  *, relu_out, din):
    p = parts_ref[...]
    nchunk = p.shape[1]
    agg = jnp.concatenate([p[0, k] + p[1, k] for k in range(nchunk)],
                          axis=-1)[:, :din]
    h = x_ref[...] + agg
    h = jnp.maximum(jnp.dot(h, w1_ref[...],
                            preferred_element_type=jnp.float32) + b1_ref[...],
                    0.0)
    o = jnp.dot(h, w2_ref[...], preferred_element_type=jnp.float32) + b2_ref[...]
    if relu_out:
        o = jnp.maximum(o, 0.0)
    o_ref[...] = o


def _gin_mlp(x, parts, w1, b1, w2, b2, relu_out):
    n, din = x.shape
    dout = w2.shape[1]
    _, nchunk, _, c_w = parts.shape
    blk = lambda d: pl.BlockSpec((ROW_BLK, d), lambda i: (i, 0))
    pblk = pl.BlockSpec((SC_CORES, nchunk, ROW_BLK, c_w),
                        lambda i: (0, 0, i, 0))
    full = lambda a: pl.BlockSpec(a.shape, lambda i: tuple(0 for _ in a.shape))
    return pl.pallas_call(
        functools.partial(_gin_mlp_body, relu_out=relu_out, din=din),
        grid=(_row_grid(n),),
        in_specs=[blk(din), pblk, full(w1), full(b1), full(w2), full(b2)],
        out_specs=blk(dout),
        out_shape=jax.ShapeDtypeStruct((n, dout), jnp.float32),
    )(x, parts, w1, b1, w2, b2)


def _gat_pre_body(x_ref, w_ref, as_ref, ad_ref, asrc_ref, adst_ref,
                  eself_ref, amax_ref, *, has_w):
    i = pl.program_id(0)
    x = x_ref[...]
    if has_w:
        avs = jnp.dot(w_ref[...], as_ref[...],
                      preferred_element_type=jnp.float32)
        avd = jnp.dot(w_ref[...], ad_ref[...],
                      preferred_element_type=jnp.float32)
    else:
        avs = as_ref[...]
        avd = ad_ref[...]
    a_s = jnp.dot(x, avs, preferred_element_type=jnp.float32)
    a_d = jnp.dot(x, avd, preferred_element_type=jnp.float32)
    asrc_ref[...] = a_s
    adst_ref[...] = a_d
    eself_ref[...] = _leaky(a_s + a_d)
    row = jax.lax.broadcasted_iota(jnp.int32, a_s.shape, 0) + i * ROW_BLK
    blk_max = jnp.max(jnp.where(row < N_NODES, a_s, -jnp.inf))

    @pl.when(i == 0)
    def _():
        amax_ref[0, 0] = -jnp.inf

    amax_ref[0, 0] = jnp.maximum(amax_ref[0, 0], blk_max)


def _gat_pre(x, w, a_s, a_d):
    n, din = x.shape
    has_w = w is not None
    if not has_w:
        w = jnp.zeros((1, 1), jnp.float32)
    blk2 = pl.BlockSpec((ROW_BLK, din), lambda i: (i, 0))
    blk1 = pl.BlockSpec((ROW_BLK,), lambda i: (i,))
    full = lambda a: pl.BlockSpec(a.shape, lambda i: tuple(0 for _ in a.shape))
    return pl.pallas_call(
        functools.partial(_gat_pre_body, has_w=has_w),
        grid=(_row_grid(n),),
        in_specs=[blk2, full(w), full(a_s), full(a_d)],
        out_specs=[blk1, blk1, blk1,
                   pl.BlockSpec((1, 1), lambda i: (0, 0),
                                memory_space=pltpu.SMEM)],
        out_shape=[jax.ShapeDtypeStruct((n,), jnp.float32)] * 3
        + [jax.ShapeDtypeStruct((1, 1), jnp.float32)],
    )(x, w, a_s, a_d)


def _gat_shift_body(adst_ref, eself_ref, amax_ref, c_ref):
    bound = _leaky(amax_ref[0, 0] + adst_ref[...])
    c_ref[...] = jnp.maximum(eself_ref[...], bound - 60.0)


def _gat_shift(adst, e_self, amax):
    n = adst.shape[0]
    blk1 = pl.BlockSpec((ROW_BLK,), lambda i: (i,))
    return pl.pallas_call(
        _gat_shift_body,
        grid=(_row_grid(n),),
        in_specs=[blk1, blk1, pl.BlockSpec((1, 1), lambda i: (0, 0))],
        out_specs=blk1,
        out_shape=jax.ShapeDtypeStruct((n,), jnp.float32),
    )(adst, e_self, amax)


def _gat_agg(wsum_ref, denom_ref, eself_ref, c_ref, x_ref):
    p = wsum_ref[...]
    nchunk = p.shape[1]
    wsum = jnp.concatenate([p[0, k] + p[1, k] for k in range(nchunk)], axis=-1)
    dp = denom_ref[...]
    denom = dp[0] + dp[1]
    ee_self = jnp.exp(eself_ref[...] - c_ref[...])
    num = wsum + ee_self[:, None] * x_ref[...]
    den = denom + ee_self + 1e-16
    return num / den[:, None]


def _gat1_post_body(wsum_ref, denom_ref, eself_ref, c_ref, x_ref, w_ref,
                    b_ref, o_ref):
    agg = _gat_agg(wsum_ref, denom_ref, eself_ref, c_ref, x_ref)
    o = jnp.dot(agg, w_ref[...], preferred_element_type=jnp.float32) + b_ref[...]
    o_ref[...] = jnp.maximum(o, 0.0)


def _gat1_post(wsum, denom, e_self, cshift, x, w, b):
    n, din = x.shape
    dout = w.shape[1]
    _, nchunk, _, c_w = wsum.shape
    blk2 = lambda d: pl.BlockSpec((ROW_BLK, d), lambda i: (i, 0))
    blk1 = pl.BlockSpec((ROW_BLK,), lambda i: (i,))
    wblk = pl.BlockSpec((SC_CORES, nchunk, ROW_BLK, c_w),
                        lambda i: (0, 0, i, 0))
    dblk = pl.BlockSpec((SC_CORES, ROW_BLK), lambda i: (0, i))
    full = lambda a: pl.BlockSpec(a.shape, lambda i: tuple(0 for _ in a.shape))
    return pl.pallas_call(
        _gat1_post_body,
        grid=(_row_grid(n),),
        in_specs=[wblk, dblk, blk1, blk1, blk2(din), full(w), full(b)],
        out_specs=blk2(dout),
        out_shape=jax.ShapeDtypeStruct((n, dout), jnp.float32),
    )(wsum, denom, e_self, cshift, x, w, b)


def _gat2_post_body(wsum_ref, denom_ref, eself_ref, c_ref, g_ref, b_ref,
                    o_ref):
    agg = _gat_agg(wsum_ref, denom_ref, eself_ref, c_ref, g_ref)
    o_ref[...] = agg + b_ref[...]


def _gat2_post(wsum, denom, e_self, cshift, g, b):
    n, d = g.shape
    _, nchunk, _, c_w = wsum.shape
    blk2 = pl.BlockSpec((ROW_BLK, d), lambda i: (i, 0))
    blk1 = pl.BlockSpec((ROW_BLK,), lambda i: (i,))
    wblk = pl.BlockSpec((SC_CORES, nchunk, ROW_BLK, c_w),
                        lambda i: (0, 0, i, 0))
    dblk = pl.BlockSpec((SC_CORES, ROW_BLK), lambda i: (0, i))
    full = pl.BlockSpec(b.shape, lambda i: (0,))
    return pl.pallas_call(
        _gat2_post_body,
        grid=(_row_grid(n),),
        in_specs=[wblk, dblk, blk1, blk1, blk2, full],
        out_specs=blk2,
        out_shape=jax.ShapeDtypeStruct((n, d), jnp.float32),
    )(wsum, denom, e_self, cshift, g, b)


def _matmul_body(x_ref, w_ref, o_ref):
    o_ref[...] = jnp.dot(x_ref[...], w_ref[...],
                         preferred_element_type=jnp.float32)


def _matmul(x, w):
    n, din = x.shape
    dout = w.shape[1]
    return pl.pallas_call(
        _matmul_body,
        grid=(_row_grid(n),),
        in_specs=[pl.BlockSpec((ROW_BLK, din), lambda i: (i, 0)),
                  pl.BlockSpec((din, dout), lambda i: (0, 0))],
        out_specs=pl.BlockSpec((ROW_BLK, dout), lambda i: (i, 0)),
        out_shape=jax.ShapeDtypeStruct((n, dout), jnp.float32),
    )(x, w)


def _prep_edges(edge_index):
    src = edge_index[0]
    dst = edge_index[1]
    pad = EPAD - E_EDGES
    src_p = jnp.concatenate([src, jnp.zeros((pad,), jnp.int32)])
    dst_p = jnp.concatenate(
        [dst, N_NODES + (jnp.arange(pad, dtype=jnp.int32) % SC_TILES)])
    return src_p.reshape(-1, EDGE_W), dst_p.reshape(-1, EDGE_W)


def _sc_segsum(tables, src2d, dst2d, zeros_nc):
    nchunk = len(tables)
    c_w = tables[0].shape[1]
    out_type = jax.ShapeDtypeStruct((SC_CORES, nchunk, NACC, c_w),
                                    jnp.float32)

    @functools.partial(
        pl.kernel, out_type=out_type, mesh=_sc_mesh(),
        compiler_params=pltpu.CompilerParams(use_tc_tiling_on_sc=False),
        scratch_types=[
            pltpu.VMEM((EDGE_W,), jnp.int32),
            pltpu.VMEM((EDGE_W,), jnp.int32),
            pltpu.VMEM((EDGE_W, c_w), jnp.float32),
            pltpu.VMEM_SHARED((NACC, c_w), jnp.float32),
        ])
    def k(*refs):
        tbls = refs[:nchunk]
        (src_hbm, dst_hbm, zeros_hbm, out_hbm,
         src_blk, dst_blk, rows, acc) = refs[nchunk:]
        cid = lax.axis_index("c")
        sid = lax.axis_index("s")
        wid = cid * SC_TILES + sid
        row0 = wid * ROWS_PER_TILE
        pltpu.sync_copy(zeros_hbm.at[pl.ds(sid * STRIPE, STRIPE)],
                        acc.at[pl.ds(sid * STRIPE, STRIPE)])
        plsc.subcore_barrier()
        for c in range(nchunk):
            @pl.loop(0, ROWS_PER_TILE)
            def _(w):
                pltpu.sync_copy(src_hbm.at[row0 + w], src_blk)
                pltpu.sync_copy(dst_hbm.at[row0 + w], dst_blk)
                pltpu.sync_copy(tbls[c].at[src_blk], rows)
                pltpu.sync_copy(rows, acc.at[dst_blk], add=True)
            plsc.subcore_barrier()
            pltpu.sync_copy(
                acc.at[pl.ds(sid * STRIPE, STRIPE)],
                out_hbm.at[cid, c, pl.ds(sid * STRIPE, STRIPE)])
            if c + 1 < nchunk:
                pltpu.sync_copy(zeros_hbm.at[pl.ds(sid * STRIPE, STRIPE)],
                                acc.at[pl.ds(sid * STRIPE, STRIPE)])
                plsc.subcore_barrier()

    return k(*tables, src2d, dst2d, zeros_nc)


def _sc_gat_edges(asrc, adst, cshift, src2d, dst2d, zeros_1):
    out_type = [
        jax.ShapeDtypeStruct((EPAD // EDGE_W, EDGE_W), jnp.float32),
        jax.ShapeDtypeStruct((SC_CORES, NACC), jnp.float32),
    ]

    @functools.partial(
        pl.kernel, out_type=out_type, mesh=_sc_mesh(),
        compiler_params=pltpu.CompilerParams(use_tc_tiling_on_sc=False),
        scratch_types=[
            pltpu.VMEM((EDGE_W,), jnp.int32),
            pltpu.VMEM((EDGE_W,), jnp.int32),
            pltpu.VMEM((EDGE_W,), jnp.float32),
            pltpu.VMEM((EDGE_W,), jnp.float32),
            pltpu.VMEM((EDGE_W,), jnp.float32),
            pltpu.VMEM((EDGE_W,), jnp.float32),
            pltpu.VMEM_SHARED((NACC,), jnp.float32),
        ])
    def k(asrc_hbm, adst_hbm, csh_hbm, src_hbm, dst_hbm, z_hbm,
          ee_hbm, den_hbm, src_blk, dst_blk, aw, bw, cw, eew, accd):
        cid = lax.axis_index("c")
        sid = lax.axis_index("s")
        wid = cid * SC_TILES + sid
        row0 = wid * ROWS_PER_TILE
        pltpu.sync_copy(z_hbm.at[pl.ds(sid * STRIPE, STRIPE)],
                        accd.at[pl.ds(sid * STRIPE, STRIPE)])
        plsc.subcore_barrier()

        @pl.loop(0, ROWS_PER_TILE)
        def _(w):
            pltpu.sync_copy(src_hbm.at[row0 + w], src_blk)
            pltpu.sync_copy(dst_hbm.at[row0 + w], dst_blk)
            pltpu.sync_copy(asrc_hbm.at[src_blk], aw)
            pltpu.sync_copy(adst_hbm.at[dst_blk], bw)
            pltpu.sync_copy(csh_hbm.at[dst_blk], cw)
            for j in range(EDGE_W // 16):
                s = pl.ds(j * 16, 16)
                t = aw[s] + bw[s]
                e = jnp.where(t > 0, t, 0.2 * t)
                eew[s] = jnp.exp(e - cw[s])
            pltpu.sync_copy(eew, ee_hbm.at[row0 + w])
            pltpu.sync_copy(eew, accd.at[dst_blk], add=True)

        plsc.subcore_barrier()
        pltpu.sync_copy(accd.at[pl.ds(sid * STRIPE, STRIPE)],
                        den_hbm.at[cid, pl.ds(sid * STRIPE, STRIPE)])

    return k(asrc, adst, cshift, src2d, dst2d, zeros_1)


def _sc_gat_wsum(tables, ee2d, src2d, dst2d, zeros_nc):
    nchunk = len(tables)
    c_w = tables[0].shape[1]
    out_type = jax.ShapeDtypeStruct((SC_CORES, nchunk, NACC, c_w),
                                    jnp.float32)

    @functools.partial(
        pl.kernel, out_type=out_type, mesh=_sc_mesh(),
        compiler_params=pltpu.CompilerParams(use_tc_tiling_on_sc=False),
        scratch_types=[
            pltpu.VMEM((EDGE_W,), jnp.int32),
            pltpu.VMEM((EDGE_W,), jnp.int32),
            pltpu.VMEM((EDGE_W,), jnp.float32),
            pltpu.VMEM((EDGE_W, c_w), jnp.float32),
            pltpu.VMEM_SHARED((NACC, c_w), jnp.float32),
        ])
    def k(*refs):
        tbls = refs[:nchunk]
        (ee_hbm, src_hbm, dst_hbm, zeros_hbm, out_hbm,
         src_blk, dst_blk, eew, rows, acc) = refs[nchunk:]
        cid = lax.axis_index("c")
        sid = lax.axis_index("s")
        wid = cid * SC_TILES + sid
        row0 = wid * ROWS_PER_TILE
        pltpu.sync_copy(zeros_hbm.at[pl.ds(sid * STRIPE, STRIPE)],
                        acc.at[pl.ds(sid * STRIPE, STRIPE)])
        plsc.subcore_barrier()
        for c in range(nchunk):
            @pl.loop(0, ROWS_PER_TILE)
            def _(w):
                pltpu.sync_copy(src_hbm.at[row0 + w], src_blk)
                pltpu.sync_copy(dst_hbm.at[row0 + w], dst_blk)
                pltpu.sync_copy(ee_hbm.at[row0 + w], eew)
                pltpu.sync_copy(tbls[c].at[src_blk], rows)
                for j in range(EDGE_W // 16):
                    ee16 = eew[pl.ds(j * 16, 16)]
                    for l in range(16):
                        i = j * 16 + l
                        b = _lane_bcast(ee16, l)
                        for cc in range(c_w // 16):
                            s = pl.ds(cc * 16, 16)
                            rows[i, s] = rows[i, s] * b
                pltpu.sync_copy(rows, acc.at[dst_blk], add=True)
            plsc.subcore_barrier()
            pltpu.sync_copy(
                acc.at[pl.ds(sid * STRIPE, STRIPE)],
                out_hbm.at[cid, c, pl.ds(sid * STRIPE, STRIPE)])
            if c + 1 < nchunk:
                pltpu.sync_copy(zeros_hbm.at[pl.ds(sid * STRIPE, STRIPE)],
                                acc.at[pl.ds(sid * STRIPE, STRIPE)])
                plsc.subcore_barrier()

    return k(*tables, ee2d, src2d, dst2d, zeros_nc)


def kernel(x, edge_index, edge_attr, gin1_w1, gin1_b1, gin1_w2, gin1_b2,
           gin2_w1, gin2_b1, gin2_w2, gin2_b2, gat1_w, gat1_asrc, gat1_adst,
           gat1_b, gat2_w, gat2_asrc, gat2_adst, gat2_b):
    del edge_attr
    n = x.shape[0]
    src = edge_index[0]
    dst = edge_index[1]
    src2d, dst2d = _prep_edges(edge_index)
    zeros16 = jnp.zeros((NACC, 16), jnp.float32)
    zeros32 = jnp.zeros((NACC, 32), jnp.float32)

    x16 = jnp.pad(x, ((0, 0), (0, 6)))
    parts = _sc_segsum([x16], src2d, dst2d, zeros16)
    h = _gin_mlp(x, parts, gin1_w1, gin1_b1, gin1_w2, gin1_b2, relu_out=True)

    parts = _sc_segsum([h[:, :32], h[:, 32:]], src2d, dst2d, zeros32)
    h = _gin_mlp(h, parts, gin2_w1, gin2_b1, gin2_w2, gin2_b2, relu_out=False)

    zeros1 = jnp.zeros((NACC,), jnp.float32)

    asrc, adst, e_self, amax = _gat_pre(h, gat1_w, gat1_asrc, gat1_adst)
    cshift = _gat_shift(adst, e_self, amax)
    ee2d, denom = _sc_gat_edges(asrc, jnp.pad(adst, (0, NACC - n)),
                                jnp.pad(cshift, (0, NACC - n)),
                                src2d, dst2d, zeros1)
    wsum = _sc_gat_wsum([h[:, k * 32:(k + 1) * 32] for k in range(4)],
                        ee2d, src2d, dst2d, zeros32)
    h = _gat1_post(wsum, denom, e_self, cshift, h, gat1_w, gat1_b)

    g = _matmul(h, gat2_w)
    asrc, adst, e_self, amax = _gat_pre(g, None, gat2_asrc, gat2_adst)
    cshift = _gat_shift(adst, e_self, amax)
    ee2d, denom = _sc_gat_edges(asrc, jnp.pad(adst, (0, NACC - n)),
                                jnp.pad(cshift, (0, NACC - n)),
                                src2d, dst2d, zeros1)
    wsum = _sc_gat_wsum([g[:, k * 32:(k + 1) * 32] for k in range(4)],
                        ee2d, src2d, dst2d, zeros32)
    return _gat2_post(wsum, denom, e_self, cshift, g, gat2_b)


# speedup vs baseline: 11.0471x; 1.4599x over previous
import functools

import jax
import jax.numpy as jnp
from jax import lax
from jax.experimental import pallas as pl
from jax.experimental.pallas import tpu as pltpu
from jax.experimental.pallas import tpu_sc as plsc

N_NODES = 50000
ROW_BLK = 2048

SC_CORES = 2
SC_TILES = 16
SC_WORKERS = SC_CORES * SC_TILES
EDGE_W = 128
E_EDGES = 800000
ROWS_PER_TILE = 200
EPAD = ROWS_PER_TILE * EDGE_W * SC_WORKERS
NACC = 50048
STRIPE = NACC // SC_TILES
G_WIN = 4
N_GRP = ROWS_PER_TILE // G_WIN

@functools.cache
def _sc_mesh():
    return plsc.VectorSubcoreMesh(core_axis_name="c", subcore_axis_name="s")


def _row_grid(n):
    return pl.cdiv(n, ROW_BLK)


def _lane_bcast(v, l):
    idx = jnp.full((16, 1), l, jnp.int32)
    dnums = lax.GatherDimensionNumbers(
        offset_dims=(), collapsed_slice_dims=(0,), start_index_map=(0,))
    return lax.gather(v, idx, dnums, (1,),
                      mode=lax.GatherScatterMode.PROMISE_IN_BOUNDS)


def _leaky(t):
    return jnp.where(t > 0, t, 0.2 * t)


def _gin_mlp_body(x_ref, parts_ref, w1_ref, b1_ref, w2_ref, b2_ref, o_ref,
                  *, relu_out, din):
    p = parts_ref[...]
    nchunk = p.shape[1]
    agg = jnp.concatenate([p[0, k] + p[1, k] for k in range(nchunk)],
                          axis=-1)[:, :din]
    h = x_ref[...] + agg
    h = jnp.maximum(jnp.dot(h, w1_ref[...],
                            preferred_element_type=jnp.float32) + b1_ref[...],
                    0.0)
    o = jnp.dot(h, w2_ref[...], preferred_element_type=jnp.float32) + b2_ref[...]
    if relu_out:
        o = jnp.maximum(o, 0.0)
    o_ref[...] = o


def _gin_mlp(x, parts, w1, b1, w2, b2, relu_out):
    n, din = x.shape
    dout = w2.shape[1]
    _, nchunk, _, c_w = parts.shape
    blk = lambda d: pl.BlockSpec((ROW_BLK, d), lambda i: (i, 0))
    pblk = pl.BlockSpec((SC_CORES, nchunk, ROW_BLK, c_w),
                        lambda i: (0, 0, i, 0))
    full = lambda a: pl.BlockSpec(a.shape, lambda i: tuple(0 for _ in a.shape))
    return pl.pallas_call(
        functools.partial(_gin_mlp_body, relu_out=relu_out, din=din),
        grid=(_row_grid(n),),
        in_specs=[blk(din), pblk, full(w1), full(b1), full(w2), full(b2)],
        out_specs=blk(dout),
        out_shape=jax.ShapeDtypeStruct((n, dout), jnp.float32),
    )(x, parts, w1, b1, w2, b2)


def _gat_pre_body(x_ref, w_ref, as_ref, ad_ref, asrc_ref, adst_ref,
                  eself_ref, amax_ref, *, has_w):
    i = pl.program_id(0)
    x = x_ref[...]
    if has_w:
        x = jnp.dot(x, w_ref[...], preferred_element_type=jnp.float32)
    a_s = jnp.dot(x, as_ref[...], preferred_element_type=jnp.float32)
    a_d = jnp.dot(x, ad_ref[...], preferred_element_type=jnp.float32)
    asrc_ref[...] = a_s
    adst_ref[...] = a_d
    eself_ref[...] = _leaky(a_s + a_d)
    row = jax.lax.broadcasted_iota(jnp.int32, a_s.shape, 0) + i * ROW_BLK
    blk_max = jnp.max(jnp.where(row < N_NODES, a_s, -jnp.inf))

    @pl.when(i == 0)
    def _():
        amax_ref[0, 0] = -jnp.inf

    amax_ref[0, 0] = jnp.maximum(amax_ref[0, 0], blk_max)


def _gat_pre(x, w, a_s, a_d):
    n, din = x.shape
    has_w = w is not None
    if not has_w:
        w = jnp.zeros((1, 1), jnp.float32)
    blk2 = pl.BlockSpec((ROW_BLK, din), lambda i: (i, 0))
    blk1 = pl.BlockSpec((ROW_BLK,), lambda i: (i,))
    full = lambda a: pl.BlockSpec(a.shape, lambda i: tuple(0 for _ in a.shape))
    return pl.pallas_call(
        functools.partial(_gat_pre_body, has_w=has_w),
        grid=(_row_grid(n),),
        in_specs=[blk2, full(w), full(a_s), full(a_d)],
        out_specs=[blk1, blk1, blk1,
                   pl.BlockSpec((1, 1), lambda i: (0, 0),
                                memory_space=pltpu.SMEM)],
        out_shape=[jax.ShapeDtypeStruct((n,), jnp.float32)] * 3
        + [jax.ShapeDtypeStruct((1, 1), jnp.float32)],
    )(x, w, a_s, a_d)


def _gat_shift_body(adst_ref, eself_ref, amax_ref, c_ref):
    bound = _leaky(amax_ref[0, 0] + adst_ref[...])
    c_ref[...] = jnp.maximum(eself_ref[...], bound - 60.0)


def _gat_shift(adst, e_self, amax):
    n = adst.shape[0]
    blk1 = pl.BlockSpec((ROW_BLK,), lambda i: (i,))
    return pl.pallas_call(
        _gat_shift_body,
        grid=(_row_grid(n),),
        in_specs=[blk1, blk1, pl.BlockSpec((1, 1), lambda i: (0, 0))],
        out_specs=blk1,
        out_shape=jax.ShapeDtypeStruct((n,), jnp.float32),
    )(adst, e_self, amax)


def _gat_agg(wsum_ref, denom_ref, eself_ref, c_ref, x_ref):
    p = wsum_ref[...]
    nchunk = p.shape[1]
    wsum = jnp.concatenate([p[0, k] + p[1, k] for k in range(nchunk)], axis=-1)
    dp = denom_ref[...]
    denom = dp[0] + dp[1]
    ee_self = jnp.exp(eself_ref[...] - c_ref[...])
    num = wsum + ee_self[:, None] * x_ref[...]
    den = denom + ee_self + 1e-16
    return num / den[:, None]


def _gat1_post_body(wsum_ref, denom_ref, eself_ref, c_ref, x_ref, w_ref,
                    b_ref, o_ref):
    agg = _gat_agg(wsum_ref, denom_ref, eself_ref, c_ref, x_ref)
    o = jnp.dot(agg, w_ref[...], preferred_element_type=jnp.float32) + b_ref[...]
    o_ref[...] = jnp.maximum(o, 0.0)


def _gat1_post(wsum, denom, e_self, cshift, x, w, b):
    n, din = x.shape
    dout = w.shape[1]
    _, nchunk, _, c_w = wsum.shape
    blk2 = lambda d: pl.BlockSpec((ROW_BLK, d), lambda i: (i, 0))
    blk1 = pl.BlockSpec((ROW_BLK,), lambda i: (i,))
    wblk = pl.BlockSpec((SC_CORES, nchunk, ROW_BLK, c_w),
                        lambda i: (0, 0, i, 0))
    dblk = pl.BlockSpec((SC_CORES, ROW_BLK), lambda i: (0, i))
    full = lambda a: pl.BlockSpec(a.shape, lambda i: tuple(0 for _ in a.shape))
    return pl.pallas_call(
        _gat1_post_body,
        grid=(_row_grid(n),),
        in_specs=[wblk, dblk, blk1, blk1, blk2(din), full(w), full(b)],
        out_specs=blk2(dout),
        out_shape=jax.ShapeDtypeStruct((n, dout), jnp.float32),
    )(wsum, denom, e_self, cshift, x, w, b)


def _gat2_post_body(wsum_ref, denom_ref, eself_ref, c_ref, g_ref, b_ref,
                    o_ref):
    agg = _gat_agg(wsum_ref, denom_ref, eself_ref, c_ref, g_ref)
    o_ref[...] = agg + b_ref[...]


def _gat2_post(wsum, denom, e_self, cshift, g, b):
    n, d = g.shape
    _, nchunk, _, c_w = wsum.shape
    blk2 = pl.BlockSpec((ROW_BLK, d), lambda i: (i, 0))
    blk1 = pl.BlockSpec((ROW_BLK,), lambda i: (i,))
    wblk = pl.BlockSpec((SC_CORES, nchunk, ROW_BLK, c_w),
                        lambda i: (0, 0, i, 0))
    dblk = pl.BlockSpec((SC_CORES, ROW_BLK), lambda i: (0, i))
    full = pl.BlockSpec(b.shape, lambda i: (0,))
    return pl.pallas_call(
        _gat2_post_body,
        grid=(_row_grid(n),),
        in_specs=[wblk, dblk, blk1, blk1, blk2, full],
        out_specs=blk2,
        out_shape=jax.ShapeDtypeStruct((n, d), jnp.float32),
    )(wsum, denom, e_self, cshift, g, b)


def _matmul_body(x_ref, w_ref, o_ref):
    o_ref[...] = jnp.dot(x_ref[...], w_ref[...],
                         preferred_element_type=jnp.float32)


def _matmul(x, w):
    n, din = x.shape
    dout = w.shape[1]
    return pl.pallas_call(
        _matmul_body,
        grid=(_row_grid(n),),
        in_specs=[pl.BlockSpec((ROW_BLK, din), lambda i: (i, 0)),
                  pl.BlockSpec((din, dout), lambda i: (0, 0))],
        out_specs=pl.BlockSpec((ROW_BLK, dout), lambda i: (i, 0)),
        out_shape=jax.ShapeDtypeStruct((n, dout), jnp.float32),
    )(x, w)


def _prep_edges(edge_index):
    src = edge_index[0]
    dst = edge_index[1]
    pad = EPAD - E_EDGES
    src_p = jnp.concatenate([src, jnp.zeros((pad,), jnp.int32)])
    dst_p = jnp.concatenate(
        [dst, N_NODES + (jnp.arange(pad, dtype=jnp.int32) % SC_TILES)])
    return src_p.reshape(-1, EDGE_W), dst_p.reshape(-1, EDGE_W)


def _sc_segsum(tables, src2d, dst2d, zeros_nc):
    nchunk = len(tables)
    c_w = tables[0].shape[1]
    out_type = jax.ShapeDtypeStruct((SC_CORES, nchunk, NACC, c_w),
                                    jnp.float32)

    @functools.partial(
        pl.kernel, out_type=out_type, mesh=_sc_mesh(),
        compiler_params=pltpu.CompilerParams(use_tc_tiling_on_sc=False),
        scratch_types=[
            pltpu.VMEM((2, G_WIN, EDGE_W), jnp.int32),
            pltpu.VMEM((2, G_WIN, EDGE_W), jnp.int32),
            pltpu.VMEM((2, EDGE_W, c_w), jnp.float32),
            pltpu.VMEM_SHARED((NACC, c_w), jnp.float32),
            pltpu.SemaphoreType.DMA,
            pltpu.SemaphoreType.DMA,
            pltpu.SemaphoreType.DMA,
            pltpu.SemaphoreType.DMA,
        ])
    def k(*refs):
        tbls = refs[:nchunk]
        (src_hbm, dst_hbm, zeros_hbm, out_hbm,
         sidx, didx, rows, acc, si0, si1, sr0, sr1) = refs[nchunk:]
        isem = (si0, si1)
        rsem = (sr0, sr1)
        cid = lax.axis_index("c")
        sid = lax.axis_index("s")
        wid = cid * SC_TILES + sid
        row0 = wid * ROWS_PER_TILE

        def idx_cp(g, par):
            s = pl.ds(row0 + g * G_WIN, G_WIN)
            return (pltpu.make_async_copy(src_hbm.at[s], sidx.at[par],
                                          isem[par]),
                    pltpu.make_async_copy(dst_hbm.at[s], didx.at[par],
                                          isem[par]))

        pltpu.sync_copy(zeros_hbm.at[pl.ds(sid * STRIPE, STRIPE)],
                        acc.at[pl.ds(sid * STRIPE, STRIPE)])
        plsc.subcore_barrier()
        for c in range(nchunk):
            tbl = tbls[c]

            def gather(par, w, rpar):
                return pltpu.make_async_copy(tbl.at[sidx.at[par, w]],
                                             rows.at[rpar], rsem[rpar])

            def group(g, par):
                @pl.when(g + 1 < N_GRP)
                def _():
                    for cp in idx_cp(g + 1, 1 - par):
                        cp.start()
                gather(par, 0, 0).start()
                for w in range(G_WIN):
                    if w + 1 < G_WIN:
                        gather(par, w + 1, (w + 1) % 2).start()
                    gather(par, w, w % 2).wait()
                    pltpu.sync_copy(rows.at[w % 2], acc.at[didx.at[par, w]],
                                    add=True)

            for cp in idx_cp(0, 0):
                cp.start()

            @pl.loop(0, N_GRP, step=2)
            def _(g):
                for cp in idx_cp(g, 0):
                    cp.wait()
                group(g, 0)
                for cp in idx_cp(g + 1, 1):
                    cp.wait()
                group(g + 1, 1)

            plsc.subcore_barrier()
            pltpu.sync_copy(
                acc.at[pl.ds(sid * STRIPE, STRIPE)],
                out_hbm.at[cid, c, pl.ds(sid * STRIPE, STRIPE)])
            if c + 1 < nchunk:
                pltpu.sync_copy(zeros_hbm.at[pl.ds(sid * STRIPE, STRIPE)],
                                acc.at[pl.ds(sid * STRIPE, STRIPE)])
                plsc.subcore_barrier()

    return k(*tables, src2d, dst2d, zeros_nc)


def _sc_gat_edges(asrc, adst, cshift, src2d, dst2d, zeros_1):
    out_type = [
        jax.ShapeDtypeStruct((EPAD // EDGE_W, EDGE_W), jnp.float32),
        jax.ShapeDtypeStruct((SC_CORES, NACC), jnp.float32),
    ]

    @functools.partial(
        pl.kernel, out_type=out_type, mesh=_sc_mesh(),
        compiler_params=pltpu.CompilerParams(use_tc_tiling_on_sc=False),
        scratch_types=[
            pltpu.VMEM((EDGE_W,), jnp.int32),
            pltpu.VMEM((EDGE_W,), jnp.int32),
            pltpu.VMEM((EDGE_W,), jnp.float32),
            pltpu.VMEM((EDGE_W,), jnp.float32),
            pltpu.VMEM((EDGE_W,), jnp.float32),
            pltpu.VMEM((EDGE_W,), jnp.float32),
            pltpu.VMEM_SHARED((NACC,), jnp.float32),
        ])
    def k(asrc_hbm, adst_hbm, csh_hbm, src_hbm, dst_hbm, z_hbm,
          ee_hbm, den_hbm, src_blk, dst_blk, aw, bw, cw, eew, accd):
        cid = lax.axis_index("c")
        sid = lax.axis_index("s")
        wid = cid * SC_TILES + sid
        row0 = wid * ROWS_PER_TILE
        pltpu.sync_copy(z_hbm.at[pl.ds(sid * STRIPE, STRIPE)],
                        accd.at[pl.ds(sid * STRIPE, STRIPE)])
        plsc.subcore_barrier()

        @pl.loop(0, ROWS_PER_TILE)
        def _(w):
            pltpu.sync_copy(src_hbm.at[row0 + w], src_blk)
            pltpu.sync_copy(dst_hbm.at[row0 + w], dst_blk)
            pltpu.sync_copy(asrc_hbm.at[src_blk], aw)
            pltpu.sync_copy(adst_hbm.at[dst_blk], bw)
            pltpu.sync_copy(csh_hbm.at[dst_blk], cw)
            for j in range(EDGE_W // 16):
                s = pl.ds(j * 16, 16)
                t = aw[s] + bw[s]
                e = jnp.where(t > 0, t, 0.2 * t)
                eew[s] = jnp.exp(e - cw[s])
            pltpu.sync_copy(eew, ee_hbm.at[row0 + w])
            pltpu.sync_copy(eew, accd.at[dst_blk], add=True)

        plsc.subcore_barrier()
        pltpu.sync_copy(accd.at[pl.ds(sid * STRIPE, STRIPE)],
                        den_hbm.at[cid, pl.ds(sid * STRIPE, STRIPE)])

    return k(asrc, adst, cshift, src2d, dst2d, zeros_1)


def _sc_gat_wsum(tables, ee2d, src2d, dst2d, zeros_nc):
    nchunk = len(tables)
    c_w = tables[0].shape[1]
    tbl3 = jnp.stack(tables)
    out_type = jax.ShapeDtypeStruct((SC_CORES, nchunk, NACC, c_w),
                                    jnp.float32)

    @functools.partial(
        pl.kernel, out_type=out_type, mesh=_sc_mesh(),
        compiler_params=pltpu.CompilerParams(use_tc_tiling_on_sc=False),
        scratch_types=[
            pltpu.VMEM((2, G_WIN, EDGE_W), jnp.int32),
            pltpu.VMEM((2, G_WIN, EDGE_W), jnp.int32),
            pltpu.VMEM((2, G_WIN, EDGE_W), jnp.float32),
            pltpu.VMEM((2, EDGE_W, c_w), jnp.float32),
            pltpu.VMEM_SHARED((NACC, c_w), jnp.float32),
            pltpu.SemaphoreType.DMA,
            pltpu.SemaphoreType.DMA,
            pltpu.SemaphoreType.DMA,
            pltpu.SemaphoreType.DMA,
        ])
    def k(tbl_hbm, ee_hbm, src_hbm, dst_hbm, zeros_hbm, out_hbm,
          sidx, didx, eeb, rows, acc, si0, si1, sr0, sr1):
        isem = (si0, si1)
        rsem = (sr0, sr1)
        cid = lax.axis_index("c")
        sid = lax.axis_index("s")
        wid = cid * SC_TILES + sid
        row0 = wid * ROWS_PER_TILE

        def idx_cp(g, par):
            s = pl.ds(row0 + g * G_WIN, G_WIN)
            return (pltpu.make_async_copy(src_hbm.at[s], sidx.at[par],
                                          isem[par]),
                    pltpu.make_async_copy(dst_hbm.at[s], didx.at[par],
                                          isem[par]),
                    pltpu.make_async_copy(ee_hbm.at[s], eeb.at[par],
                                          isem[par]))

        pltpu.sync_copy(zeros_hbm.at[pl.ds(sid * STRIPE, STRIPE)],
                        acc.at[pl.ds(sid * STRIPE, STRIPE)])
        plsc.subcore_barrier()

        @pl.loop(0, nchunk)
        def _(c):
            def gather(par, w, rpar):
                return pltpu.make_async_copy(
                    tbl_hbm.at[c].at[sidx.at[par, w]],
                    rows.at[rpar], rsem[rpar])

            def do_window(par, w, rpar):
                gather(par, w, rpar).wait()
                for j in range(EDGE_W // 16):
                    ee16 = eeb[par, w, pl.ds(j * 16, 16)]
                    for l in range(16):
                        i = j * 16 + l
                        b = _lane_bcast(ee16, l)
                        for cc in range(c_w // 16):
                            s = pl.ds(cc * 16, 16)
                            rows[rpar, i, s] = rows[rpar, i, s] * b
                pltpu.sync_copy(rows.at[rpar], acc.at[didx.at[par, w]],
                                add=True)

            def group(g, par):
                @pl.when(g + 1 < N_GRP)
                def _():
                    for cp in idx_cp(g + 1, 1 - par):
                        cp.start()
                gather(par, 0, 0).start()
                gather(par, 1, 1).start()

                for w in range(0, G_WIN, 2):
                    do_window(par, w, 0)
                    if w + 2 < G_WIN:
                        gather(par, w + 2, 0).start()
                    do_window(par, w + 1, 1)
                    if w + 3 < G_WIN:
                        gather(par, w + 3, 1).start()

            for cp in idx_cp(0, 0):
                cp.start()

            @pl.loop(0, N_GRP, step=2)
            def _(g):
                for cp in idx_cp(g, 0):
                    cp.wait()
                group(g, 0)
                for cp in idx_cp(g + 1, 1):
                    cp.wait()
                group(g + 1, 1)

            plsc.subcore_barrier()
            pltpu.sync_copy(
                acc.at[pl.ds(sid * STRIPE, STRIPE)],
                out_hbm.at[cid, c, pl.ds(sid * STRIPE, STRIPE)])

            @pl.when(c + 1 < nchunk)
            def _():
                pltpu.sync_copy(zeros_hbm.at[pl.ds(sid * STRIPE, STRIPE)],
                                acc.at[pl.ds(sid * STRIPE, STRIPE)])

            plsc.subcore_barrier()

    return k(tbl3, ee2d, src2d, dst2d, zeros_nc)


def kernel(x, edge_index, edge_attr, gin1_w1, gin1_b1, gin1_w2, gin1_b2,
           gin2_w1, gin2_b1, gin2_w2, gin2_b2, gat1_w, gat1_asrc, gat1_adst,
           gat1_b, gat2_w, gat2_asrc, gat2_adst, gat2_b):
    del edge_attr
    n = x.shape[0]
    src = edge_index[0]
    dst = edge_index[1]
    src2d, dst2d = _prep_edges(edge_index)
    zeros16 = jnp.zeros((NACC, 16), jnp.float32)
    zeros32 = jnp.zeros((NACC, 32), jnp.float32)

    x16 = jnp.pad(x, ((0, 0), (0, 6)))
    parts = _sc_segsum([x16], src2d, dst2d, zeros16)
    h = _gin_mlp(x, parts, gin1_w1, gin1_b1, gin1_w2, gin1_b2, relu_out=True)

    parts = _sc_segsum([h[:, :32], h[:, 32:]], src2d, dst2d, zeros32)
    h = _gin_mlp(h, parts, gin2_w1, gin2_b1, gin2_w2, gin2_b2, relu_out=False)

    zeros1 = jnp.zeros((NACC,), jnp.float32)

    asrc, adst, e_self, amax = _gat_pre(h, gat1_w, gat1_asrc, gat1_adst)
    cshift = _gat_shift(adst, e_self, amax)
    ee2d, denom = _sc_gat_edges(asrc, jnp.pad(adst, (0, NACC - n)),
                                jnp.pad(cshift, (0, NACC - n)),
                                src2d, dst2d, zeros1)
    wsum = _sc_gat_wsum([h[:, k * 32:(k + 1) * 32] for k in range(4)],
                        ee2d, src2d, dst2d, zeros32)
    h = _gat1_post(wsum, denom, e_self, cshift, h, gat1_w, gat1_b)

    g = _matmul(h, gat2_w)
    asrc, adst, e_self, amax = _gat_pre(g, None, gat2_asrc, gat2_adst)
    cshift = _gat_shift(adst, e_self, amax)
    ee2d, denom = _sc_gat_edges(asrc, jnp.pad(adst, (0, NACC - n)),
                                jnp.pad(cshift, (0, NACC - n)),
                                src2d, dst2d, zeros1)
    wsum = _sc_gat_wsum([g[:, k * 32:(k + 1) * 32] for k in range(4)],
                        ee2d, src2d, dst2d, zeros32)
    return _gat2_post(wsum, denom, e_self, cshift, g, gat2_b)


# speedup vs baseline: 12.6818x; 1.1480x over previous
import functools

import jax
import jax.numpy as jnp
from jax import lax
from jax.experimental import pallas as pl
from jax.experimental.pallas import tpu as pltpu
from jax.experimental.pallas import tpu_sc as plsc

N_NODES = 50000
ROW_BLK = 2048

SC_CORES = 2
SC_TILES = 16
SC_WORKERS = SC_CORES * SC_TILES
EDGE_W = 128
E_EDGES = 800000
ROWS_PER_TILE = 200
EPAD = ROWS_PER_TILE * EDGE_W * SC_WORKERS
NACC = 50048
STRIPE = NACC // SC_TILES
G_WIN = 4
N_GRP = ROWS_PER_TILE // G_WIN

@functools.cache
def _sc_mesh():
    return plsc.VectorSubcoreMesh(core_axis_name="c", subcore_axis_name="s")


def _row_grid(n):
    return pl.cdiv(n, ROW_BLK)


def _lane_bcast(v, l):
    idx = jnp.full((16, 1), l, jnp.int32)
    dnums = lax.GatherDimensionNumbers(
        offset_dims=(), collapsed_slice_dims=(0,), start_index_map=(0,))
    return lax.gather(v, idx, dnums, (1,),
                      mode=lax.GatherScatterMode.PROMISE_IN_BOUNDS)


def _leaky(t):
    return jnp.where(t > 0, t, 0.2 * t)


def _gin_mlp_body(x_ref, parts_ref, w1_ref, b1_ref, w2_ref, b2_ref, o_ref,
                  *, relu_out, din):
    p = parts_ref[...]
    nchunk = p.shape[1]
    agg = jnp.concatenate([p[0, k] + p[1, k] for k in range(nchunk)],
                          axis=-1)[:, :din]
    h = x_ref[...] + agg
    h = jnp.maximum(jnp.dot(h, w1_ref[...],
                            preferred_element_type=jnp.float32) + b1_ref[...],
                    0.0)
    o = jnp.dot(h, w2_ref[...], preferred_element_type=jnp.float32) + b2_ref[...]
    if relu_out:
        o = jnp.maximum(o, 0.0)
    o_ref[...] = o


def _gin_mlp(x, parts, w1, b1, w2, b2, relu_out):
    n, din = x.shape
    dout = w2.shape[1]
    _, nchunk, _, c_w = parts.shape
    blk = lambda d: pl.BlockSpec((ROW_BLK, d), lambda i: (i, 0))
    pblk = pl.BlockSpec((SC_CORES, nchunk, ROW_BLK, c_w),
                        lambda i: (0, 0, i, 0))
    full = lambda a: pl.BlockSpec(a.shape, lambda i: tuple(0 for _ in a.shape))
    return pl.pallas_call(
        functools.partial(_gin_mlp_body, relu_out=relu_out, din=din),
        grid=(_row_grid(n),),
        in_specs=[blk(din), pblk, full(w1), full(b1), full(w2), full(b2)],
        out_specs=blk(dout),
        out_shape=jax.ShapeDtypeStruct((n, dout), jnp.float32),
    )(x, parts, w1, b1, w2, b2)


def _gat_pre_body(x_ref, w_ref, as_ref, ad_ref, asrc_ref, adst_ref,
                  eself_ref, amax_ref, *, has_w):
    i = pl.program_id(0)
    x = x_ref[...]
    if has_w:
        x = jnp.dot(x, w_ref[...], preferred_element_type=jnp.float32)
    a_s = jnp.dot(x, as_ref[...], preferred_element_type=jnp.float32)
    a_d = jnp.dot(x, ad_ref[...], preferred_element_type=jnp.float32)
    asrc_ref[...] = a_s
    adst_ref[...] = a_d
    eself_ref[...] = _leaky(a_s + a_d)
    row = jax.lax.broadcasted_iota(jnp.int32, a_s.shape, 0) + i * ROW_BLK
    blk_max = jnp.max(jnp.where(row < N_NODES, a_s, -jnp.inf))

    @pl.when(i == 0)
    def _():
        amax_ref[0, 0] = -jnp.inf

    amax_ref[0, 0] = jnp.maximum(amax_ref[0, 0], blk_max)


def _gat_pre(x, w, a_s, a_d):
    n, din = x.shape
    has_w = w is not None
    if not has_w:
        w = jnp.zeros((1, 1), jnp.float32)
    blk2 = pl.BlockSpec((ROW_BLK, din), lambda i: (i, 0))
    blk1 = pl.BlockSpec((ROW_BLK,), lambda i: (i,))
    full = lambda a: pl.BlockSpec(a.shape, lambda i: tuple(0 for _ in a.shape))
    return pl.pallas_call(
        functools.partial(_gat_pre_body, has_w=has_w),
        grid=(_row_grid(n),),
        in_specs=[blk2, full(w), full(a_s), full(a_d)],
        out_specs=[blk1, blk1, blk1,
                   pl.BlockSpec((1, 1), lambda i: (0, 0),
                                memory_space=pltpu.SMEM)],
        out_shape=[jax.ShapeDtypeStruct((n,), jnp.float32)] * 3
        + [jax.ShapeDtypeStruct((1, 1), jnp.float32)],
    )(x, w, a_s, a_d)


def _gat_shift_body(adst_ref, eself_ref, amax_ref, c_ref):
    bound = _leaky(amax_ref[0, 0] + adst_ref[...])
    c_ref[...] = jnp.maximum(eself_ref[...], bound - 60.0)


def _gat_shift(adst, e_self, amax):
    n = adst.shape[0]
    blk1 = pl.BlockSpec((ROW_BLK,), lambda i: (i,))
    return pl.pallas_call(
        _gat_shift_body,
        grid=(_row_grid(n),),
        in_specs=[blk1, blk1, pl.BlockSpec((1, 1), lambda i: (0, 0))],
        out_specs=blk1,
        out_shape=jax.ShapeDtypeStruct((n,), jnp.float32),
    )(adst, e_self, amax)


def _gat_agg(wsum_ref, denom_ref, eself_ref, c_ref, x_ref):
    p = wsum_ref[...]
    nchunk = p.shape[1]
    wsum = jnp.concatenate([p[0, k] + p[1, k] for k in range(nchunk)], axis=-1)
    dp = denom_ref[...]
    denom = dp[0] + dp[1]
    ee_self = jnp.exp(eself_ref[...] - c_ref[...])
    num = wsum + ee_self[:, None] * x_ref[...]
    den = denom + ee_self + 1e-16
    return num / den[:, None]


def _gat1_post_body(wsum_ref, denom_ref, eself_ref, c_ref, x_ref, w_ref,
                    b_ref, o_ref):
    agg = _gat_agg(wsum_ref, denom_ref, eself_ref, c_ref, x_ref)
    o = jnp.dot(agg, w_ref[...], preferred_element_type=jnp.float32) + b_ref[...]
    o_ref[...] = jnp.maximum(o, 0.0)


def _gat1_post(wsum, denom, e_self, cshift, x, w, b):
    n, din = x.shape
    dout = w.shape[1]
    _, nchunk, _, c_w = wsum.shape
    blk2 = lambda d: pl.BlockSpec((ROW_BLK, d), lambda i: (i, 0))
    blk1 = pl.BlockSpec((ROW_BLK,), lambda i: (i,))
    wblk = pl.BlockSpec((SC_CORES, nchunk, ROW_BLK, c_w),
                        lambda i: (0, 0, i, 0))
    dblk = pl.BlockSpec((SC_CORES, ROW_BLK), lambda i: (0, i))
    full = lambda a: pl.BlockSpec(a.shape, lambda i: tuple(0 for _ in a.shape))
    return pl.pallas_call(
        _gat1_post_body,
        grid=(_row_grid(n),),
        in_specs=[wblk, dblk, blk1, blk1, blk2(din), full(w), full(b)],
        out_specs=blk2(dout),
        out_shape=jax.ShapeDtypeStruct((n, dout), jnp.float32),
    )(wsum, denom, e_self, cshift, x, w, b)


def _gat2_post_body(wsum_ref, denom_ref, eself_ref, c_ref, g_ref, b_ref,
                    o_ref):
    agg = _gat_agg(wsum_ref, denom_ref, eself_ref, c_ref, g_ref)
    o_ref[...] = agg + b_ref[...]


def _gat2_post(wsum, denom, e_self, cshift, g, b):
    n, d = g.shape
    _, nchunk, _, c_w = wsum.shape
    blk2 = pl.BlockSpec((ROW_BLK, d), lambda i: (i, 0))
    blk1 = pl.BlockSpec((ROW_BLK,), lambda i: (i,))
    wblk = pl.BlockSpec((SC_CORES, nchunk, ROW_BLK, c_w),
                        lambda i: (0, 0, i, 0))
    dblk = pl.BlockSpec((SC_CORES, ROW_BLK), lambda i: (0, i))
    full = pl.BlockSpec(b.shape, lambda i: (0,))
    return pl.pallas_call(
        _gat2_post_body,
        grid=(_row_grid(n),),
        in_specs=[wblk, dblk, blk1, blk1, blk2, full],
        out_specs=blk2,
        out_shape=jax.ShapeDtypeStruct((n, d), jnp.float32),
    )(wsum, denom, e_self, cshift, g, b)


def _matmul_body(x_ref, w_ref, o_ref):
    o_ref[...] = jnp.dot(x_ref[...], w_ref[...],
                         preferred_element_type=jnp.float32)


def _matmul(x, w):
    n, din = x.shape
    dout = w.shape[1]
    return pl.pallas_call(
        _matmul_body,
        grid=(_row_grid(n),),
        in_specs=[pl.BlockSpec((ROW_BLK, din), lambda i: (i, 0)),
                  pl.BlockSpec((din, dout), lambda i: (0, 0))],
        out_specs=pl.BlockSpec((ROW_BLK, dout), lambda i: (i, 0)),
        out_shape=jax.ShapeDtypeStruct((n, dout), jnp.float32),
    )(x, w)


def _prep_edges(edge_index):
    src = edge_index[0]
    dst = edge_index[1]
    pad = EPAD - E_EDGES
    src_p = jnp.concatenate([src, jnp.zeros((pad,), jnp.int32)])
    dst_p = jnp.concatenate(
        [dst, N_NODES + (jnp.arange(pad, dtype=jnp.int32) % SC_TILES)])
    return src_p.reshape(-1, EDGE_W), dst_p.reshape(-1, EDGE_W)


def _sc_segsum(tables, src2d, dst2d, zeros_nc):
    nchunk = len(tables)
    c_w = tables[0].shape[1]
    out_type = jax.ShapeDtypeStruct((SC_CORES, nchunk, NACC, c_w),
                                    jnp.float32)

    @functools.partial(
        pl.kernel, out_type=out_type, mesh=_sc_mesh(),
        compiler_params=pltpu.CompilerParams(use_tc_tiling_on_sc=False),
        scratch_types=[
            pltpu.VMEM((2, G_WIN, EDGE_W), jnp.int32),
            pltpu.VMEM((2, G_WIN, EDGE_W), jnp.int32),
            pltpu.VMEM((2, EDGE_W, c_w), jnp.float32),
            pltpu.VMEM_SHARED((NACC, c_w), jnp.float32),
            pltpu.SemaphoreType.DMA,
            pltpu.SemaphoreType.DMA,
            pltpu.SemaphoreType.DMA,
            pltpu.SemaphoreType.DMA,
        ])
    def k(*refs):
        tbls = refs[:nchunk]
        (src_hbm, dst_hbm, zeros_hbm, out_hbm,
         sidx, didx, rows, acc, si0, si1, sr0, sr1) = refs[nchunk:]
        isem = (si0, si1)
        rsem = (sr0, sr1)
        cid = lax.axis_index("c")
        sid = lax.axis_index("s")
        wid = cid * SC_TILES + sid
        row0 = wid * ROWS_PER_TILE

        def idx_cp(g, par):
            s = pl.ds(row0 + g * G_WIN, G_WIN)
            return (pltpu.make_async_copy(src_hbm.at[s], sidx.at[par],
                                          isem[par]),
                    pltpu.make_async_copy(dst_hbm.at[s], didx.at[par],
                                          isem[par]))

        pltpu.sync_copy(zeros_hbm.at[pl.ds(sid * STRIPE, STRIPE)],
                        acc.at[pl.ds(sid * STRIPE, STRIPE)])
        plsc.subcore_barrier()
        for c in range(nchunk):
            tbl = tbls[c]

            def gather(par, w, rpar):
                return pltpu.make_async_copy(tbl.at[sidx.at[par, w]],
                                             rows.at[rpar], rsem[rpar])

            def group(g, par):
                @pl.when(g + 1 < N_GRP)
                def _():
                    for cp in idx_cp(g + 1, 1 - par):
                        cp.start()
                gather(par, 0, 0).start()
                for w in range(G_WIN):
                    if w + 1 < G_WIN:
                        gather(par, w + 1, (w + 1) % 2).start()
                    gather(par, w, w % 2).wait()
                    pltpu.sync_copy(rows.at[w % 2], acc.at[didx.at[par, w]],
                                    add=True)

            for cp in idx_cp(0, 0):
                cp.start()

            @pl.loop(0, N_GRP, step=2)
            def _(g):
                for cp in idx_cp(g, 0):
                    cp.wait()
                group(g, 0)
                for cp in idx_cp(g + 1, 1):
                    cp.wait()
                group(g + 1, 1)

            plsc.subcore_barrier()
            pltpu.sync_copy(
                acc.at[pl.ds(sid * STRIPE, STRIPE)],
                out_hbm.at[cid, c, pl.ds(sid * STRIPE, STRIPE)])
            if c + 1 < nchunk:
                pltpu.sync_copy(zeros_hbm.at[pl.ds(sid * STRIPE, STRIPE)],
                                acc.at[pl.ds(sid * STRIPE, STRIPE)])
                plsc.subcore_barrier()

    return k(*tables, src2d, dst2d, zeros_nc)


def _sc_gat_edges(asrc, adst, cshift, src2d, dst2d, zeros_1):
    out_type = [
        jax.ShapeDtypeStruct((EPAD // EDGE_W, EDGE_W), jnp.float32),
        jax.ShapeDtypeStruct((SC_CORES, NACC), jnp.float32),
    ]

    @functools.partial(
        pl.kernel, out_type=out_type, mesh=_sc_mesh(),
        compiler_params=pltpu.CompilerParams(use_tc_tiling_on_sc=False),
        scratch_types=[
            pltpu.VMEM((2, G_WIN, EDGE_W), jnp.int32),
            pltpu.VMEM((2, G_WIN, EDGE_W), jnp.int32),
            pltpu.VMEM((2, EDGE_W), jnp.float32),
            pltpu.VMEM((2, EDGE_W), jnp.float32),
            pltpu.VMEM((2, EDGE_W), jnp.float32),
            pltpu.VMEM((2, G_WIN, EDGE_W), jnp.float32),
            pltpu.VMEM_SHARED((NACC,), jnp.float32),
            pltpu.SemaphoreType.DMA,
            pltpu.SemaphoreType.DMA,
            pltpu.SemaphoreType.DMA,
            pltpu.SemaphoreType.DMA,
        ])
    def k(asrc_hbm, adst_hbm, csh_hbm, src_hbm, dst_hbm, z_hbm,
          ee_hbm, den_hbm, sidx, didx, aw, bw, cw, eeg, accd,
          si0, si1, sg0, sg1):
        isem = (si0, si1)
        gsem = (sg0, sg1)
        cid = lax.axis_index("c")
        sid = lax.axis_index("s")
        wid = cid * SC_TILES + sid
        row0 = wid * ROWS_PER_TILE

        def idx_cp(g, par):
            s = pl.ds(row0 + g * G_WIN, G_WIN)
            return (pltpu.make_async_copy(src_hbm.at[s], sidx.at[par],
                                          isem[par]),
                    pltpu.make_async_copy(dst_hbm.at[s], didx.at[par],
                                          isem[par]))

        def gath(par, w, rpar):
            return (pltpu.make_async_copy(asrc_hbm.at[sidx.at[par, w]],
                                          aw.at[rpar], gsem[rpar]),
                    pltpu.make_async_copy(adst_hbm.at[didx.at[par, w]],
                                          bw.at[rpar], gsem[rpar]),
                    pltpu.make_async_copy(csh_hbm.at[didx.at[par, w]],
                                          cw.at[rpar], gsem[rpar]))

        pltpu.sync_copy(z_hbm.at[pl.ds(sid * STRIPE, STRIPE)],
                        accd.at[pl.ds(sid * STRIPE, STRIPE)])
        plsc.subcore_barrier()

        def egroup(g, par):
            @pl.when(g + 1 < N_GRP)
            def _():
                for cp in idx_cp(g + 1, 1 - par):
                    cp.start()
            for cp in gath(par, 0, 0):
                cp.start()
            for w in range(G_WIN):
                if w + 1 < G_WIN:
                    for cp in gath(par, w + 1, (w + 1) % 2):
                        cp.start()
                for cp in gath(par, w, w % 2):
                    cp.wait()
                rp = w % 2
                for j in range(EDGE_W // 16):
                    s = pl.ds(j * 16, 16)
                    t = aw[rp, s] + bw[rp, s]
                    e = jnp.where(t > 0, t, 0.2 * t)
                    eeg[par, w, s] = jnp.exp(e - cw[rp, s])
                pltpu.sync_copy(eeg.at[par, w], accd.at[didx.at[par, w]],
                                add=True)
            pltpu.sync_copy(eeg.at[par],
                            ee_hbm.at[pl.ds(row0 + g * G_WIN, G_WIN)])

        for cp in idx_cp(0, 0):
            cp.start()

        @pl.loop(0, N_GRP, step=2)
        def _(g):
            for cp in idx_cp(g, 0):
                cp.wait()
            egroup(g, 0)
            for cp in idx_cp(g + 1, 1):
                cp.wait()
            egroup(g + 1, 1)

        plsc.subcore_barrier()
        pltpu.sync_copy(accd.at[pl.ds(sid * STRIPE, STRIPE)],
                        den_hbm.at[cid, pl.ds(sid * STRIPE, STRIPE)])

    return k(asrc, adst, cshift, src2d, dst2d, zeros_1)


def _sc_gat_wsum(tables, ee2d, src2d, dst2d, zeros_nc):
    nchunk = len(tables)
    c_w = tables[0].shape[1]
    tbl3 = jnp.stack(tables)
    out_type = jax.ShapeDtypeStruct((SC_CORES, nchunk, NACC, c_w),
                                    jnp.float32)

    @functools.partial(
        pl.kernel, out_type=out_type, mesh=_sc_mesh(),
        compiler_params=pltpu.CompilerParams(use_tc_tiling_on_sc=False),
        scratch_types=[
            pltpu.VMEM((2, G_WIN, EDGE_W), jnp.int32),
            pltpu.VMEM((2, G_WIN, EDGE_W), jnp.int32),
            pltpu.VMEM((2, G_WIN, EDGE_W), jnp.float32),
            pltpu.VMEM((2, EDGE_W, c_w), jnp.float32),
            pltpu.VMEM_SHARED((NACC, c_w), jnp.float32),
            pltpu.SemaphoreType.DMA,
            pltpu.SemaphoreType.DMA,
            pltpu.SemaphoreType.DMA,
            pltpu.SemaphoreType.DMA,
        ])
    def k(tbl_hbm, ee_hbm, src_hbm, dst_hbm, zeros_hbm, out_hbm,
          sidx, didx, eeb, rows, acc, si0, si1, sr0, sr1):
        isem = (si0, si1)
        rsem = (sr0, sr1)
        cid = lax.axis_index("c")
        sid = lax.axis_index("s")
        wid = cid * SC_TILES + sid
        row0 = wid * ROWS_PER_TILE

        def idx_cp(g, par):
            s = pl.ds(row0 + g * G_WIN, G_WIN)
            return (pltpu.make_async_copy(src_hbm.at[s], sidx.at[par],
                                          isem[par]),
                    pltpu.make_async_copy(dst_hbm.at[s], didx.at[par],
                                          isem[par]),
                    pltpu.make_async_copy(ee_hbm.at[s], eeb.at[par],
                                          isem[par]))

        pltpu.sync_copy(zeros_hbm.at[pl.ds(sid * STRIPE, STRIPE)],
                        acc.at[pl.ds(sid * STRIPE, STRIPE)])
        plsc.subcore_barrier()

        @pl.loop(0, nchunk)
        def _(c):
            def gather(par, w, rpar):
                return pltpu.make_async_copy(
                    tbl_hbm.at[c].at[sidx.at[par, w]],
                    rows.at[rpar], rsem[rpar])

            def do_window(par, w, rpar):
                gather(par, w, rpar).wait()
                for j in range(EDGE_W // 16):
                    ee16 = eeb[par, w, pl.ds(j * 16, 16)]
                    for l in range(16):
                        i = j * 16 + l
                        b = _lane_bcast(ee16, l)
                        for cc in range(c_w // 16):
                            s = pl.ds(cc * 16, 16)
                            rows[rpar, i, s] = rows[rpar, i, s] * b
                pltpu.sync_copy(rows.at[rpar], acc.at[didx.at[par, w]],
                                add=True)

            def group(g, par):
                @pl.when(g + 1 < N_GRP)
                def _():
                    for cp in idx_cp(g + 1, 1 - par):
                        cp.start()
                gather(par, 0, 0).start()
                gather(par, 1, 1).start()

                for w in range(0, G_WIN, 2):
                    do_window(par, w, 0)
                    if w + 2 < G_WIN:
                        gather(par, w + 2, 0).start()
                    do_window(par, w + 1, 1)
                    if w + 3 < G_WIN:
                        gather(par, w + 3, 1).start()

            for cp in idx_cp(0, 0):
                cp.start()

            @pl.loop(0, N_GRP, step=2)
            def _(g):
                for cp in idx_cp(g, 0):
                    cp.wait()
                group(g, 0)
                for cp in idx_cp(g + 1, 1):
                    cp.wait()
                group(g + 1, 1)

            plsc.subcore_barrier()
            pltpu.sync_copy(
                acc.at[pl.ds(sid * STRIPE, STRIPE)],
                out_hbm.at[cid, c, pl.ds(sid * STRIPE, STRIPE)])

            @pl.when(c + 1 < nchunk)
            def _():
                pltpu.sync_copy(zeros_hbm.at[pl.ds(sid * STRIPE, STRIPE)],
                                acc.at[pl.ds(sid * STRIPE, STRIPE)])

            plsc.subcore_barrier()

    return k(tbl3, ee2d, src2d, dst2d, zeros_nc)


def kernel(x, edge_index, edge_attr, gin1_w1, gin1_b1, gin1_w2, gin1_b2,
           gin2_w1, gin2_b1, gin2_w2, gin2_b2, gat1_w, gat1_asrc, gat1_adst,
           gat1_b, gat2_w, gat2_asrc, gat2_adst, gat2_b):
    del edge_attr
    n = x.shape[0]
    src = edge_index[0]
    dst = edge_index[1]
    src2d, dst2d = _prep_edges(edge_index)
    zeros16 = jnp.zeros((NACC, 16), jnp.float32)
    zeros32 = jnp.zeros((NACC, 32), jnp.float32)

    x16 = jnp.pad(x, ((0, 0), (0, 6)))
    parts = _sc_segsum([x16], src2d, dst2d, zeros16)
    h = _gin_mlp(x, parts, gin1_w1, gin1_b1, gin1_w2, gin1_b2, relu_out=True)

    parts = _sc_segsum([h[:, :32], h[:, 32:]], src2d, dst2d, zeros32)
    h = _gin_mlp(h, parts, gin2_w1, gin2_b1, gin2_w2, gin2_b2, relu_out=False)

    zeros1 = jnp.zeros((NACC,), jnp.float32)

    asrc, adst, e_self, amax = _gat_pre(h, gat1_w, gat1_asrc, gat1_adst)
    cshift = _gat_shift(adst, e_self, amax)
    ee2d, denom = _sc_gat_edges(asrc, jnp.pad(adst, (0, NACC - n)),
                                jnp.pad(cshift, (0, NACC - n)),
                                src2d, dst2d, zeros1)
    wsum = _sc_gat_wsum([h[:, k * 32:(k + 1) * 32] for k in range(4)],
                        ee2d, src2d, dst2d, zeros32)
    h = _gat1_post(wsum, denom, e_self, cshift, h, gat1_w, gat1_b)

    g = _matmul(h, gat2_w)
    asrc, adst, e_self, amax = _gat_pre(g, None, gat2_asrc, gat2_adst)
    cshift = _gat_shift(adst, e_self, amax)
    ee2d, denom = _sc_gat_edges(asrc, jnp.pad(adst, (0, NACC - n)),
                                jnp.pad(cshift, (0, NACC - n)),
                                src2d, dst2d, zeros1)
    wsum = _sc_gat_wsum([g[:, k * 32:(k + 1) * 32] for k in range(4)],
                        ee2d, src2d, dst2d, zeros32)
    return _gat2_post(wsum, denom, e_self, cshift, g, gat2_b)
